# Initial kernel scaffold; baseline (speedup 1.0000x reference)
import jax
import jax.numpy as jnp
from jax.experimental import pallas as pl


def kernel(n_mu, n_sigma2, e_mu, e_sigma2, batch_node_key, batch_node_value, batch_edge_key, batch_edge_value):
    raise NotImplementedError("write your pallas kernel here")


# speedup vs baseline: 1.2598x; 1.2598x over previous
import functools

import jax
import jax.numpy as jnp
from jax import lax
from jax.experimental import pallas as pl
from jax.experimental.pallas import tpu as pltpu
from jax.experimental.pallas import tpu_sc as plsc

_EPS = 1.0
_LAMB = 0.5
_N_NODES = 50000
_N_EDGES = 1600000

_NW = 32
_E_PER_W = _N_EDGES // _NW
_ECH = 10000
_ECHUNKS = _E_PER_W // _ECH
_NODE_WORKERS = 25
_NCH = _N_NODES // _NODE_WORKERS


def _nll_partial(mu_b, s2_b, v0_b, v1_b, nvec, acc):
    def body(j, a):
        o = j * 16
        mu = mu_b[pl.ds(o, 16)]
        s2 = s2_b[pl.ds(o, 16)]
        v0 = v0_b[pl.ds(o, 16)]
        v1 = v1_b[pl.ds(o, 16)]
        x = s2 + _EPS
        t = s2 / (s2 + 2.0)
        t2 = t * t
        lg = t * (2.0 + t2 * (2.0 / 3.0 + t2 * (2.0 / 5.0 + t2 * (2.0 / 7.0 + t2 * (2.0 / 9.0)))))
        d = v0 - mu
        return a + (0.5 * lg + d * d / x) * v1

    return lax.fori_loop(0, nvec, body, acc)


def _sc_body(n_mu, n_s2, e_mu, e_s2, nkey, nval0, nval1, ekey, eval0, eval1,
             out_node, out_edge,
             ekey_b, emu_b, es2_b, ev0_b, ev1_b,
             nkey_b, nmu_b, ns2_b, nv0_b, nv1_b,
             stage_b, sem0, sem1, sem2, sem3):
    cid = lax.axis_index("c")
    sid = lax.axis_index("s")
    wid = sid * 2 + cid

    def echunk(c, acc):
        base = pl.multiple_of(wid * _E_PER_W + c * _ECH, 8)
        cpv0 = pltpu.async_copy(eval0.at[pl.ds(base, _ECH)], ev0_b, sem2)
        cpv1 = pltpu.async_copy(eval1.at[pl.ds(base, _ECH)], ev1_b, sem3)
        pltpu.sync_copy(ekey.at[pl.ds(base, _ECH)], ekey_b)
        cp0 = pltpu.async_copy(e_mu.at[ekey_b], emu_b, sem0)
        cp1 = pltpu.async_copy(e_s2.at[ekey_b], es2_b, sem1)
        cpv0.wait()
        cpv1.wait()
        cp0.wait()
        cp1.wait()
        return _nll_partial(emu_b, es2_b, ev0_b, ev1_b, _ECH // 16, acc)

    eacc = lax.fori_loop(0, _ECHUNKS, echunk, jnp.zeros((16,), jnp.float32))
    stage_b[...] = eacc
    pltpu.sync_copy(stage_b, out_edge.at[wid])

    stage_b[...] = jnp.zeros((16,), jnp.float32)

    @pl.when(wid < _NODE_WORKERS)
    def _():
        base = pl.multiple_of(wid * _NCH, 8)
        cpv0 = pltpu.async_copy(nval0.at[pl.ds(base, _NCH)], nv0_b, sem2)
        cpv1 = pltpu.async_copy(nval1.at[pl.ds(base, _NCH)], nv1_b, sem3)
        pltpu.sync_copy(nkey.at[pl.ds(base, _NCH)], nkey_b)
        cp0 = pltpu.async_copy(n_mu.at[nkey_b], nmu_b, sem0)
        cp1 = pltpu.async_copy(n_s2.at[nkey_b], ns2_b, sem1)
        cpv0.wait()
        cpv1.wait()
        cp0.wait()
        cp1.wait()
        nacc = _nll_partial(nmu_b, ns2_b, nv0_b, nv1_b, _NCH // 16,
                            jnp.zeros((16,), jnp.float32))
        stage_b[...] = nacc

    pltpu.sync_copy(stage_b, out_node.at[wid])


_sc_kernel = pl.kernel(
    _sc_body,
    out_type=(jax.ShapeDtypeStruct((_NW, 16), jnp.float32),
              jax.ShapeDtypeStruct((_NW, 16), jnp.float32)),
    mesh=plsc.VectorSubcoreMesh(core_axis_name="c", subcore_axis_name="s"),
    scratch_types=[
        pltpu.VMEM((_ECH,), jnp.int32),
        pltpu.VMEM((_ECH,), jnp.float32),
        pltpu.VMEM((_ECH,), jnp.float32),
        pltpu.VMEM((_ECH,), jnp.float32),
        pltpu.VMEM((_ECH,), jnp.float32),
        pltpu.VMEM((_NCH,), jnp.int32),
        pltpu.VMEM((_NCH,), jnp.float32),
        pltpu.VMEM((_NCH,), jnp.float32),
        pltpu.VMEM((_NCH,), jnp.float32),
        pltpu.VMEM((_NCH,), jnp.float32),
        pltpu.VMEM((16,), jnp.float32),
        pltpu.SemaphoreType.DMA,
        pltpu.SemaphoreType.DMA,
        pltpu.SemaphoreType.DMA,
        pltpu.SemaphoreType.DMA,
    ],
)


def _combine_body(np_ref, ep_ref, on_ref, oe_ref, ot_ref):
    n = jnp.sum(np_ref[...]) * (1.0 / _N_NODES)
    e = jnp.sum(ep_ref[...]) * (1.0 / _N_EDGES)
    on_ref[0, 0] = n
    oe_ref[0, 0] = e
    ot_ref[0, 0] = n * _LAMB + e * (1.0 - _LAMB)


_combine = pl.pallas_call(
    _combine_body,
    out_shape=(jax.ShapeDtypeStruct((1, 1), jnp.float32),
               jax.ShapeDtypeStruct((1, 1), jnp.float32),
               jax.ShapeDtypeStruct((1, 1), jnp.float32)),
    out_specs=(pl.BlockSpec(memory_space=pltpu.SMEM),
               pl.BlockSpec(memory_space=pltpu.SMEM),
               pl.BlockSpec(memory_space=pltpu.SMEM)),
)


def kernel(n_mu, n_sigma2, e_mu, e_sigma2, batch_node_key, batch_node_value,
           batch_edge_key, batch_edge_value):
    node_pp, edge_pp = _sc_kernel(
        n_mu, n_sigma2, e_mu, e_sigma2,
        batch_node_key.astype(jnp.int32),
        batch_node_value[:, 0], batch_node_value[:, 1],
        batch_edge_key.astype(jnp.int32),
        batch_edge_value[:, 0], batch_edge_value[:, 1])
    on, oe, ot = _combine(node_pp, edge_pp)
    return (on[0, 0], oe[0, 0], ot[0, 0])
